# baseline (speedup 1.0000x reference)
import jax
import jax.numpy as jnp
from jax import lax
from jax.experimental import pallas as pl
from jax.experimental.pallas import tpu as pltpu
from jax.experimental.pallas import tpu_sc as plsc

VOCAB = 1_000_000
EMBED = 64
SEQ = 200
BATCH = 4096
B = BATCH * SEQ

_info = plsc.get_sparse_core_info()
NC, NS = _info.num_cores, _info.num_subcores
NW = NC * NS

PANEL = 128
MAIN_COLS = 999936
NPANEL = MAIN_COLS // PANEL
PANELS_BASE = NPANEL // NW
PANELS_EXTRA = NPANEL - PANELS_BASE * NW


def _panel_transpose(a_v, b_v):
    rows16 = lax.iota(jnp.int32, 16)
    for e in range(PANEL):
        dst_base = (e % 2) * EMBED
        for v in range(EMBED // 16):
            vals = plsc.load_gather(
                a_v, [16 * v + rows16, jnp.full((16,), e, jnp.int32)]
            )
            b_v[e // 2, pl.ds(dst_base + 16 * v, 16)] = vals


def _transpose_body(tt_hbm, tail_hbm, t2_hbm, a0, b0, a1, b1, tl, rs0, rs1, ws0, ws1):
    wid = lax.axis_index("s") * NC + lax.axis_index("c")
    base = PANELS_BASE * wid + jnp.minimum(wid, PANELS_EXTRA)
    count = PANELS_BASE + jnp.where(wid < PANELS_EXTRA, 1, 0)
    last = base + count - 1

    def pidx(t):
        return jnp.minimum(base + t, last)

    def read_start(p, buf, sem):
        pltpu.async_copy(tt_hbm.at[:, pl.ds(p * PANEL, PANEL)], buf, sem)

    def read_wait(p, buf, sem):
        pltpu.make_async_copy(
            tt_hbm.at[:, pl.ds(p * PANEL, PANEL)], buf, sem
        ).wait()

    def write_start(p, buf, sem):
        pltpu.async_copy(buf, t2_hbm.at[pl.ds(p * (PANEL // 2), PANEL // 2)], sem)

    def write_wait(p, buf, sem):
        pltpu.make_async_copy(
            buf, t2_hbm.at[pl.ds(p * (PANEL // 2), PANEL // 2)], sem
        ).wait()

    read_start(pidx(0), a0, rs0)
    read_start(pidx(1), a1, rs1)

    def pair(t, carry):
        p0 = pidx(2 * t)
        p1 = pidx(2 * t + 1)
        read_wait(p0, a0, rs0)
        _panel_transpose(a0, b0)
        write_start(p0, b0, ws0)
        read_wait(p1, a1, rs1)
        _panel_transpose(a1, b1)
        write_start(p1, b1, ws1)
        read_start(pidx(2 * t + 2), a0, rs0)
        read_start(pidx(2 * t + 3), a1, rs1)
        write_wait(p0, b0, ws0)
        write_wait(p1, b1, ws1)
        return carry

    npairs = (count + 1) // 2
    lax.fori_loop(0, npairs, pair, 0)
    read_wait(last, a0, rs0)
    read_wait(last, a1, rs1)

    @pl.when(wid == 0)
    def _():
        pltpu.sync_copy(tail_hbm, tl)
        pltpu.sync_copy(tl, t2_hbm.at[pl.ds(MAIN_COLS // 2, 32)])


@jax.jit
def _format_table(table_t, tail):
    mesh = plsc.VectorSubcoreMesh(core_axis_name="c", subcore_axis_name="s")
    return pl.kernel(
        _transpose_body,
        out_type=jax.ShapeDtypeStruct((VOCAB // 2, 2 * EMBED), jnp.float32),
        mesh=mesh,
        scratch_types=[
            pltpu.VMEM((EMBED, PANEL), jnp.float32),
            pltpu.VMEM((PANEL // 2, 2 * EMBED), jnp.float32),
            pltpu.VMEM((EMBED, PANEL), jnp.float32),
            pltpu.VMEM((PANEL // 2, 2 * EMBED), jnp.float32),
            pltpu.VMEM((32, 2 * EMBED), jnp.float32),
            pltpu.SemaphoreType.DMA,
            pltpu.SemaphoreType.DMA,
            pltpu.SemaphoreType.DMA,
            pltpu.SemaphoreType.DMA,
        ],
        compiler_params=pltpu.CompilerParams(
            use_tc_tiling_on_sc=True, needs_layout_passes=False
        ),
    )(table_t, tail)


B_PER_W = B // NW
CHUNK = 800
NCHUNK = B_PER_W // CHUNK


def _gather_body(table_hbm, x_hbm, out_hbm, idx_v, rows0, rows1, gs0, gs1, ss0, ss1):
    wid = lax.axis_index("s") * NC + lax.axis_index("c")
    base = wid * B_PER_W
    pltpu.sync_copy(x_hbm.at[pl.ds(base, B_PER_W)], idx_v)

    def gather_start(g, buf, sem):
        pltpu.async_copy(
            table_hbm.at[idx_v.at[pl.ds(g * CHUNK, CHUNK)]], buf, sem
        )

    def gather_wait(g, buf, sem):
        pltpu.make_async_copy(
            table_hbm.at[idx_v.at[pl.ds(g * CHUNK, CHUNK)]], buf, sem
        ).wait()

    def scatter_start(g, buf, sem):
        pltpu.async_copy(
            buf,
            out_hbm.at[pl.ds(base + g * CHUNK, CHUNK), pl.ds(0, EMBED)],
            sem,
        )

    def scatter_wait(g, buf, sem):
        pltpu.make_async_copy(
            buf,
            out_hbm.at[pl.ds(base + g * CHUNK, CHUNK), pl.ds(0, EMBED)],
            sem,
        ).wait()

    gather_start(0, rows0, gs0)
    gather_start(1, rows1, gs1)

    def pair(t, carry):
        g0 = t * 2
        gather_wait(g0, rows0, gs0)
        scatter_start(g0, rows0, ss0)
        gather_wait(g0 + 1, rows1, gs1)
        scatter_start(g0 + 1, rows1, ss1)
        scatter_wait(g0, rows0, ss0)
        gather_start(g0 + 2, rows0, gs0)
        scatter_wait(g0 + 1, rows1, ss1)
        gather_start(g0 + 3, rows1, gs1)
        return carry

    lax.fori_loop(0, NCHUNK // 2 - 1, pair, 0)

    last = NCHUNK - 2
    gather_wait(last, rows0, gs0)
    scatter_start(last, rows0, ss0)
    gather_wait(last + 1, rows1, gs1)
    scatter_start(last + 1, rows1, ss1)
    scatter_wait(last, rows0, ss0)
    scatter_wait(last + 1, rows1, ss1)


@jax.jit
def _lookup(x_flat, table_lin):
    mesh = plsc.VectorSubcoreMesh(core_axis_name="c", subcore_axis_name="s")
    return pl.kernel(
        _gather_body,
        out_type=jax.ShapeDtypeStruct((B, 2 * EMBED), jnp.float32),
        mesh=mesh,
        scratch_types=[
            pltpu.VMEM((B_PER_W,), jnp.int32),
            pltpu.VMEM((CHUNK, EMBED), jnp.float32),
            pltpu.VMEM((CHUNK, EMBED), jnp.float32),
            pltpu.SemaphoreType.DMA,
            pltpu.SemaphoreType.DMA,
            pltpu.SemaphoreType.DMA,
            pltpu.SemaphoreType.DMA,
        ],
        compiler_params=pltpu.CompilerParams(use_tc_tiling_on_sc=False),
    )(table_lin, x_flat)


def kernel(x, table):
    table_t = table.T
    tail = table[MAIN_COLS:].reshape(32, 2 * EMBED)
    t2 = _format_table(table_t, tail)
    out_p = _lookup(x.reshape(B), t2.reshape(VOCAB, EMBED))
    return out_p[:, :EMBED].reshape(BATCH, SEQ, EMBED)


# speedup vs baseline: 1.0007x; 1.0007x over previous
import jax
import jax.numpy as jnp
from jax import lax
from jax.experimental import pallas as pl
from jax.experimental.pallas import tpu as pltpu
from jax.experimental.pallas import tpu_sc as plsc

VOCAB = 1_000_000
EMBED = 64
SEQ = 200
BATCH = 4096
B = BATCH * SEQ

_info = plsc.get_sparse_core_info()
NC, NS = _info.num_cores, _info.num_subcores
NW = NC * NS

PANEL = 128
MAIN_COLS = 999936
NPANEL = MAIN_COLS // PANEL
PANELS_BASE = NPANEL // NW
PANELS_EXTRA = NPANEL - PANELS_BASE * NW


def _panel_transpose(a_v, b_v):
    rows16 = lax.iota(jnp.int32, 16)
    for e in range(PANEL):
        dst_base = (e % 2) * EMBED
        for v in range(EMBED // 16):
            vals = plsc.load_gather(
                a_v, [16 * v + rows16, jnp.full((16,), e, jnp.int32)]
            )
            b_v[e // 2, pl.ds(dst_base + 16 * v, 16)] = vals


def _transpose_body(tt_hbm, tail_hbm, t2_hbm, a0, b0, a1, b1, tl, rs0, rs1, ws0, ws1):
    wid = lax.axis_index("s") * NC + lax.axis_index("c")
    base = PANELS_BASE * wid + jnp.minimum(wid, PANELS_EXTRA)
    count = PANELS_BASE + jnp.where(wid < PANELS_EXTRA, 1, 0)
    last = base + count - 1

    def pidx(t):
        return jnp.minimum(base + t, last)

    def read_start(p, buf, sem):
        pltpu.async_copy(tt_hbm.at[:, pl.ds(p * PANEL, PANEL)], buf, sem)

    def read_wait(p, buf, sem):
        pltpu.make_async_copy(
            tt_hbm.at[:, pl.ds(p * PANEL, PANEL)], buf, sem
        ).wait()

    def write_start(p, buf, sem):
        pltpu.async_copy(buf, t2_hbm.at[pl.ds(p * (PANEL // 2), PANEL // 2)], sem)

    def write_wait(p, buf, sem):
        pltpu.make_async_copy(
            buf, t2_hbm.at[pl.ds(p * (PANEL // 2), PANEL // 2)], sem
        ).wait()

    read_start(pidx(0), a0, rs0)
    read_start(pidx(1), a1, rs1)

    def pair(t, carry):
        p0 = pidx(2 * t)
        p1 = pidx(2 * t + 1)
        read_wait(p0, a0, rs0)
        _panel_transpose(a0, b0)
        write_start(p0, b0, ws0)
        read_wait(p1, a1, rs1)
        _panel_transpose(a1, b1)
        write_start(p1, b1, ws1)
        read_start(pidx(2 * t + 2), a0, rs0)
        read_start(pidx(2 * t + 3), a1, rs1)
        write_wait(p0, b0, ws0)
        write_wait(p1, b1, ws1)
        return carry

    npairs = (count + 1) // 2
    lax.fori_loop(0, npairs, pair, 0)
    read_wait(last, a0, rs0)
    read_wait(last, a1, rs1)

    @pl.when(wid == 0)
    def _():
        pltpu.sync_copy(tail_hbm, tl)
        pltpu.sync_copy(tl, t2_hbm.at[pl.ds(MAIN_COLS // 2, 32)])


@jax.jit
def _format_table(table_t, tail):
    mesh = plsc.VectorSubcoreMesh(core_axis_name="c", subcore_axis_name="s")
    return pl.kernel(
        _transpose_body,
        out_type=jax.ShapeDtypeStruct((VOCAB // 2, 2 * EMBED), jnp.float32),
        mesh=mesh,
        scratch_types=[
            pltpu.VMEM((EMBED, PANEL), jnp.float32),
            pltpu.VMEM((PANEL // 2, 2 * EMBED), jnp.float32),
            pltpu.VMEM((EMBED, PANEL), jnp.float32),
            pltpu.VMEM((PANEL // 2, 2 * EMBED), jnp.float32),
            pltpu.VMEM((32, 2 * EMBED), jnp.float32),
            pltpu.SemaphoreType.DMA,
            pltpu.SemaphoreType.DMA,
            pltpu.SemaphoreType.DMA,
            pltpu.SemaphoreType.DMA,
        ],
        compiler_params=pltpu.CompilerParams(
            use_tc_tiling_on_sc=True,
            needs_layout_passes=False,
            disable_bounds_checks=True,
        ),
    )(table_t, tail)


B_PER_W = B // NW
CHUNK = 800
NCHUNK = B_PER_W // CHUNK


def _gather_body(table_hbm, x_hbm, out_hbm, idx_v, rows0, rows1, gs0, gs1, ss0, ss1):
    wid = lax.axis_index("s") * NC + lax.axis_index("c")
    base = wid * B_PER_W
    pltpu.sync_copy(x_hbm.at[pl.ds(base, B_PER_W)], idx_v)

    def gather_start(g, buf, sem):
        pltpu.async_copy(
            table_hbm.at[idx_v.at[pl.ds(g * CHUNK, CHUNK)]], buf, sem
        )

    def gather_wait(g, buf, sem):
        pltpu.make_async_copy(
            table_hbm.at[idx_v.at[pl.ds(g * CHUNK, CHUNK)]], buf, sem
        ).wait()

    def scatter_start(g, buf, sem):
        pltpu.async_copy(
            buf,
            out_hbm.at[pl.ds(base + g * CHUNK, CHUNK), pl.ds(0, EMBED)],
            sem,
        )

    def scatter_wait(g, buf, sem):
        pltpu.make_async_copy(
            buf,
            out_hbm.at[pl.ds(base + g * CHUNK, CHUNK), pl.ds(0, EMBED)],
            sem,
        ).wait()

    gather_start(0, rows0, gs0)
    gather_start(1, rows1, gs1)

    def pair(t, carry):
        g0 = t * 2
        gather_wait(g0, rows0, gs0)
        scatter_start(g0, rows0, ss0)
        gather_wait(g0 + 1, rows1, gs1)
        scatter_start(g0 + 1, rows1, ss1)
        scatter_wait(g0, rows0, ss0)
        gather_start(g0 + 2, rows0, gs0)
        scatter_wait(g0 + 1, rows1, ss1)
        gather_start(g0 + 3, rows1, gs1)
        return carry

    lax.fori_loop(0, NCHUNK // 2 - 1, pair, 0)

    last = NCHUNK - 2
    gather_wait(last, rows0, gs0)
    scatter_start(last, rows0, ss0)
    gather_wait(last + 1, rows1, gs1)
    scatter_start(last + 1, rows1, ss1)
    scatter_wait(last, rows0, ss0)
    scatter_wait(last + 1, rows1, ss1)


@jax.jit
def _lookup(x_flat, table_lin):
    mesh = plsc.VectorSubcoreMesh(core_axis_name="c", subcore_axis_name="s")
    return pl.kernel(
        _gather_body,
        out_type=jax.ShapeDtypeStruct((B, 2 * EMBED), jnp.float32),
        mesh=mesh,
        scratch_types=[
            pltpu.VMEM((B_PER_W,), jnp.int32),
            pltpu.VMEM((CHUNK, EMBED), jnp.float32),
            pltpu.VMEM((CHUNK, EMBED), jnp.float32),
            pltpu.SemaphoreType.DMA,
            pltpu.SemaphoreType.DMA,
            pltpu.SemaphoreType.DMA,
            pltpu.SemaphoreType.DMA,
        ],
        compiler_params=pltpu.CompilerParams(use_tc_tiling_on_sc=False),
    )(table_lin, x_flat)


def kernel(x, table):
    table_t = table.T
    tail = table[MAIN_COLS:].reshape(32, 2 * EMBED)
    t2 = _format_table(table_t, tail)
    out_p = _lookup(x.reshape(B), t2.reshape(VOCAB, EMBED))
    return out_p[:, :EMBED].reshape(BATCH, SEQ, EMBED)


# speedup vs baseline: 2.3543x; 2.3527x over previous
import jax
import jax.numpy as jnp
from jax import lax
from jax.experimental import pallas as pl
from jax.experimental.pallas import tpu as pltpu
from jax.experimental.pallas import tpu_sc as plsc

VOCAB = 1_000_000
EMBED = 64
SEQ = 200
BATCH = 4096
B = BATCH * SEQ

_info = plsc.get_sparse_core_info()
NC, NS = _info.num_cores, _info.num_subcores
NW = NC * NS
B_PER_W = B // NW
CHUNK = 800
NCHUNK = B_PER_W // CHUNK


def _gather_body(table_hbm, x_hbm, out_hbm, idx_v, rows0, rows1, gs0, gs1, ss0, ss1):
    wid = lax.axis_index("s") * NC + lax.axis_index("c")
    base = wid * B_PER_W
    pltpu.sync_copy(x_hbm.at[pl.ds(base, B_PER_W)], idx_v)

    def gather_start(g, buf, sem):
        pltpu.async_copy(
            table_hbm.at[idx_v.at[pl.ds(g * CHUNK, CHUNK)]], buf, sem
        )

    def gather_wait(g, buf, sem):
        pltpu.make_async_copy(
            table_hbm.at[idx_v.at[pl.ds(g * CHUNK, CHUNK)]], buf, sem
        ).wait()

    def scatter_start(g, buf, sem):
        pltpu.async_copy(
            buf,
            out_hbm.at[pl.ds(base + g * CHUNK, CHUNK), pl.ds(0, EMBED)],
            sem,
        )

    def scatter_wait(g, buf, sem):
        pltpu.make_async_copy(
            buf,
            out_hbm.at[pl.ds(base + g * CHUNK, CHUNK), pl.ds(0, EMBED)],
            sem,
        ).wait()

    gather_start(0, rows0, gs0)
    gather_start(1, rows1, gs1)

    def pair(t, carry):
        g0 = t * 2
        gather_wait(g0, rows0, gs0)
        scatter_start(g0, rows0, ss0)
        gather_wait(g0 + 1, rows1, gs1)
        scatter_start(g0 + 1, rows1, ss1)
        scatter_wait(g0, rows0, ss0)
        gather_start(g0 + 2, rows0, gs0)
        scatter_wait(g0 + 1, rows1, ss1)
        gather_start(g0 + 3, rows1, gs1)
        return carry

    lax.fori_loop(0, NCHUNK // 2 - 1, pair, 0)

    last = NCHUNK - 2
    gather_wait(last, rows0, gs0)
    scatter_start(last, rows0, ss0)
    gather_wait(last + 1, rows1, gs1)
    scatter_start(last + 1, rows1, ss1)
    scatter_wait(last, rows0, ss0)
    scatter_wait(last + 1, rows1, ss1)


@jax.jit
def _lookup(x_flat, table_lin):
    mesh = plsc.VectorSubcoreMesh(core_axis_name="c", subcore_axis_name="s")
    return pl.kernel(
        _gather_body,
        out_type=jax.ShapeDtypeStruct((B, 2 * EMBED), jnp.float32),
        mesh=mesh,
        scratch_types=[
            pltpu.VMEM((B_PER_W,), jnp.int32),
            pltpu.VMEM((CHUNK, EMBED), jnp.float32),
            pltpu.VMEM((CHUNK, EMBED), jnp.float32),
            pltpu.SemaphoreType.DMA,
            pltpu.SemaphoreType.DMA,
            pltpu.SemaphoreType.DMA,
            pltpu.SemaphoreType.DMA,
        ],
        compiler_params=pltpu.CompilerParams(use_tc_tiling_on_sc=False),
    )(table_lin, x_flat)


def kernel(x, table):
    out_p = _lookup(x.reshape(B), table)
    return out_p[:, :EMBED].reshape(BATCH, SEQ, EMBED)


# speedup vs baseline: 2.3704x; 1.0068x over previous
import jax
import jax.numpy as jnp
from jax import lax
from jax.experimental import pallas as pl
from jax.experimental.pallas import tpu as pltpu
from jax.experimental.pallas import tpu_sc as plsc

VOCAB = 1_000_000
EMBED = 64
SEQ = 200
BATCH = 4096
B = BATCH * SEQ

_info = plsc.get_sparse_core_info()
NC, NS = _info.num_cores, _info.num_subcores
NW = NC * NS
B_PER_W = B // NW
CHUNK = 400
NCHUNK = B_PER_W // CHUNK
NBUF = 4
NQUAD = NCHUNK // NBUF


def _gather_body(table_hbm, x_hbm, out_hbm, idx_v, b0, b1, b2, b3,
                 gs0, gs1, gs2, gs3, ss0, ss1, ss2, ss3):
    wid = lax.axis_index("s") * NC + lax.axis_index("c")
    base = wid * B_PER_W
    pltpu.sync_copy(x_hbm.at[pl.ds(base, B_PER_W)], idx_v)

    bufs = (b0, b1, b2, b3)
    gsems = (gs0, gs1, gs2, gs3)
    ssems = (ss0, ss1, ss2, ss3)

    def gather_start(g, buf, sem):
        pltpu.async_copy(
            table_hbm.at[idx_v.at[pl.ds(g * CHUNK, CHUNK)]], buf, sem
        )

    def gather_wait(g, buf, sem):
        pltpu.make_async_copy(
            table_hbm.at[idx_v.at[pl.ds(g * CHUNK, CHUNK)]], buf, sem
        ).wait()

    def scatter_start(g, buf, sem):
        pltpu.async_copy(
            buf,
            out_hbm.at[pl.ds(base + g * CHUNK, CHUNK), pl.ds(0, EMBED)],
            sem,
        )

    def scatter_wait(g, buf, sem):
        pltpu.make_async_copy(
            buf,
            out_hbm.at[pl.ds(base + g * CHUNK, CHUNK), pl.ds(0, EMBED)],
            sem,
        ).wait()

    for k in range(NBUF):
        gather_start(k, bufs[k], gsems[k])

    def quad(t, carry):
        g0 = t * NBUF
        for k in range(NBUF):
            gather_wait(g0 + k, bufs[k], gsems[k])
            scatter_start(g0 + k, bufs[k], ssems[k])
        for k in range(NBUF):
            scatter_wait(g0 + k, bufs[k], ssems[k])
            gather_start(g0 + NBUF + k, bufs[k], gsems[k])
        return carry

    lax.fori_loop(0, NQUAD - 1, quad, 0)

    last = (NQUAD - 1) * NBUF
    for k in range(NBUF):
        gather_wait(last + k, bufs[k], gsems[k])
        scatter_start(last + k, bufs[k], ssems[k])
    for k in range(NBUF):
        scatter_wait(last + k, bufs[k], ssems[k])


@jax.jit
def _lookup(x_flat, table_lin):
    mesh = plsc.VectorSubcoreMesh(core_axis_name="c", subcore_axis_name="s")
    return pl.kernel(
        _gather_body,
        out_type=jax.ShapeDtypeStruct((B, 2 * EMBED), jnp.float32),
        mesh=mesh,
        scratch_types=[
            pltpu.VMEM((B_PER_W,), jnp.int32),
            pltpu.VMEM((CHUNK, EMBED), jnp.float32),
            pltpu.VMEM((CHUNK, EMBED), jnp.float32),
            pltpu.VMEM((CHUNK, EMBED), jnp.float32),
            pltpu.VMEM((CHUNK, EMBED), jnp.float32),
            pltpu.SemaphoreType.DMA,
            pltpu.SemaphoreType.DMA,
            pltpu.SemaphoreType.DMA,
            pltpu.SemaphoreType.DMA,
            pltpu.SemaphoreType.DMA,
            pltpu.SemaphoreType.DMA,
            pltpu.SemaphoreType.DMA,
            pltpu.SemaphoreType.DMA,
        ],
        compiler_params=pltpu.CompilerParams(use_tc_tiling_on_sc=False),
    )(table_lin, x_flat)


def kernel(x, table):
    out_p = _lookup(x.reshape(B), table)
    return out_p[:, :EMBED].reshape(BATCH, SEQ, EMBED)
